# baseline (speedup 1.0000x reference)
import functools

import jax
import jax.numpy as jnp
from jax import lax
from jax.experimental import pallas as pl
from jax.experimental.pallas import tpu as pltpu
from jax.experimental.pallas import tpu_sc as plsc

BATCH = 16384
HIST = 50
DIM = 32

NC = 2
NS = 16
NW = NC * NS
BPW = BATCH // NW
NTJ = BPW // 128
SUB = 64
NSUB = BPW // SUB
D1 = 40

_mesh = plsc.VectorSubcoreMesh(core_axis_name="c", subcore_axis_name="s")


@functools.partial(
    pl.kernel,
    mesh=_mesh,
    out_type=jax.ShapeDtypeStruct((HIST, DIM // 8, BATCH // 128, 8, 128),
                                  jnp.float32),
    scratch_types=[
        pltpu.VMEM((2, BPW), jnp.int32),
        pltpu.VMEM((2, BPW, DIM), jnp.float32),
        pltpu.VMEM((2, DIM // 8, D1, 129), jnp.float32),
        pltpu.SemaphoreType.DMA,
        pltpu.SemaphoreType.DMA,
        pltpu.SemaphoreType.DMA,
        pltpu.SemaphoreType.DMA,
        pltpu.SemaphoreType.DMA,
        pltpu.SemaphoreType.DMA,
    ],
    compiler_params=pltpu.CompilerParams(use_tc_tiling_on_sc=False,
                                         needs_layout_passes=False),
)
def _sc_gather(idx_hbm, table_hbm, out_hbm, idx_v, rows_v, tr_v,
               sem_g0, sem_g1, sem_i0, sem_i1, sem_o0, sem_o1):
    wid = lax.axis_index("s") * NC + lax.axis_index("c")
    base = wid * BPW
    j0 = wid * NTJ
    sem_g = (sem_g0, sem_g1)
    sem_i = (sem_i0, sem_i1)
    sem_o = (sem_o0, sem_o1)
    iota16 = lax.iota(jnp.int32, 16)
    zeros16 = jnp.zeros((16,), jnp.int32)
    i_vecs = [iota16 // 8 + 2 * m for m in range(2)]
    s_vecs = [iota16 & 7] * 2

    def issue_gathers(p):
        for j in range(NSUB):
            pltpu.async_copy(table_hbm.at[idx_v.at[p].at[pl.ds(j * SUB, SUB)]],
                             rows_v.at[p].at[pl.ds(j * SUB, SUB)], sem_g[p])

    def wait_gathers(p):
        for j in range(NSUB):
            pltpu.make_async_copy(
                table_hbm.at[idx_v.at[p].at[pl.ds(j * SUB, SUB)]],
                rows_v.at[p].at[pl.ds(j * SUB, SUB)], sem_g[p]).wait()

    def issue_idx(h, p):
        pltpu.async_copy(idx_hbm.at[h, pl.ds(base, BPW)], idx_v.at[p],
                         sem_i[p])

    def wait_idx(p):
        pltpu.make_async_copy(idx_hbm.at[0, pl.ds(base, BPW)], idx_v.at[p],
                              sem_i[p]).wait()

    def issue_out(h, p):
        for i in range(DIM // 8):
            for jj in range(NTJ):
                pltpu.async_copy(
                    tr_v.at[p, i, pl.ds(jj * 8, 8), pl.ds(0, 128)],
                    out_hbm.at[h, i, j0 + jj], sem_o[p])

    def wait_out(p):
        for i in range(DIM // 8):
            for jj in range(NTJ):
                pltpu.make_async_copy(
                    tr_v.at[p, i, pl.ds(jj * 8, 8), pl.ds(0, 128)],
                    out_hbm.at[0, i, j0 + jj], sem_o[p]).wait()

    def transpose(p):
        rows = rows_v.at[p]
        tr = tr_v.at[p]

        def body(bi, carry):
            for u in range(4):
                b = bi * 4 + u
                d1s = (b // 128) * 8
                l_vec = (b % 128) + zeros16
                for m in range(2):
                    v = rows[b, pl.ds(16 * m, 16)]
                    plsc.store_scatter(tr, [i_vecs[m], s_vecs[m] + d1s,
                                            l_vec], v)
            return carry

        lax.fori_loop(0, BPW // 4, body, 0)

    def half(h, p, first=False, last=False):
        q = 1 - p
        if not last:
            wait_idx(q)
            issue_gathers(q)
        wait_gathers(p)
        if isinstance(h, int) and h + 2 <= HIST - 1:
            issue_idx(h + 2, p)
        elif not isinstance(h, int):
            issue_idx(h + 2, p)
        if not first:
            wait_out(p)
        transpose(p)
        issue_out(h, p)

    pltpu.sync_copy(idx_hbm.at[0, pl.ds(base, BPW)], idx_v.at[0])
    issue_gathers(0)
    issue_idx(1, 1)

    half(0, 0, first=True)
    half(1, 1, first=True)

    def body2(i, carry):
        h = 2 * i + 2
        half(h, 0)
        half(h + 1, 1)
        return carry

    lax.fori_loop(0, (HIST - 4) // 2, body2, 0)

    half(HIST - 2, 0)
    half(HIST - 1, 1, last=True)
    wait_out(0)
    wait_out(1)


def kernel(sets, E):
    out5 = _sc_gather(sets.T, E)
    return out5.transpose(2, 4, 0, 1, 3).reshape(BATCH, HIST, DIM)


# speedup vs baseline: 1.0064x; 1.0064x over previous
import functools

import jax
import jax.numpy as jnp
from jax import lax
from jax.experimental import pallas as pl
from jax.experimental.pallas import tpu as pltpu
from jax.experimental.pallas import tpu_sc as plsc

BATCH = 16384
HIST = 50
DIM = 32

NC = 2
NS = 16
NW = NC * NS
BPW = BATCH // NW
NTJ = BPW // 128
SUB = 128
NSUB = BPW // SUB
D1 = 40

_mesh = plsc.VectorSubcoreMesh(core_axis_name="c", subcore_axis_name="s")


@functools.partial(
    pl.kernel,
    mesh=_mesh,
    out_type=jax.ShapeDtypeStruct((HIST, DIM // 8, BATCH // 128, 8, 128),
                                  jnp.float32),
    scratch_types=[
        pltpu.VMEM((2, BPW), jnp.int32),
        pltpu.VMEM((2, BPW, DIM), jnp.float32),
        pltpu.VMEM((2, DIM // 8, D1, 129), jnp.float32),
        pltpu.SemaphoreType.DMA,
        pltpu.SemaphoreType.DMA,
        pltpu.SemaphoreType.DMA,
        pltpu.SemaphoreType.DMA,
        pltpu.SemaphoreType.DMA,
        pltpu.SemaphoreType.DMA,
    ],
    compiler_params=pltpu.CompilerParams(use_tc_tiling_on_sc=False,
                                         needs_layout_passes=False),
)
def _sc_gather(idx_hbm, table_hbm, out_hbm, idx_v, rows_v, tr_v,
               sem_g0, sem_g1, sem_i0, sem_i1, sem_o0, sem_o1):
    wid = lax.axis_index("s") * NC + lax.axis_index("c")
    base = wid * BPW
    j0 = wid * NTJ
    sem_g = (sem_g0, sem_g1)
    sem_i = (sem_i0, sem_i1)
    sem_o = (sem_o0, sem_o1)
    iota16 = lax.iota(jnp.int32, 16)
    zeros16 = jnp.zeros((16,), jnp.int32)
    i_vecs = [iota16 // 8 + 2 * m for m in range(2)]
    s_vecs = [iota16 & 7] * 2

    def issue_gathers(p):
        for j in range(NSUB):
            pltpu.async_copy(table_hbm.at[idx_v.at[p].at[pl.ds(j * SUB, SUB)]],
                             rows_v.at[p].at[pl.ds(j * SUB, SUB)], sem_g[p])

    def wait_gathers(p):
        for j in range(NSUB):
            pltpu.make_async_copy(
                table_hbm.at[idx_v.at[p].at[pl.ds(j * SUB, SUB)]],
                rows_v.at[p].at[pl.ds(j * SUB, SUB)], sem_g[p]).wait()

    def issue_idx(h, p):
        pltpu.async_copy(idx_hbm.at[h, pl.ds(base, BPW)], idx_v.at[p],
                         sem_i[p])

    def wait_idx(p):
        pltpu.make_async_copy(idx_hbm.at[0, pl.ds(base, BPW)], idx_v.at[p],
                              sem_i[p]).wait()

    def issue_out(h, p):
        for i in range(DIM // 8):
            for jj in range(NTJ):
                pltpu.async_copy(
                    tr_v.at[p, i, pl.ds(jj * 8, 8), pl.ds(0, 128)],
                    out_hbm.at[h, i, j0 + jj], sem_o[p])

    def wait_out(p):
        for i in range(DIM // 8):
            for jj in range(NTJ):
                pltpu.make_async_copy(
                    tr_v.at[p, i, pl.ds(jj * 8, 8), pl.ds(0, 128)],
                    out_hbm.at[0, i, j0 + jj], sem_o[p]).wait()

    def transpose(p):
        rows = rows_v.at[p]
        tr = tr_v.at[p]

        def body(bi, carry):
            b0 = bi * 16
            d1s = (b0 // 128) * 8
            l0 = b0 % 128
            for u in range(16):
                b = b0 + u
                l_vec = (l0 + u) + zeros16
                for m in range(2):
                    v = rows[b, pl.ds(16 * m, 16)]
                    plsc.store_scatter(tr, [i_vecs[m], s_vecs[m] + d1s,
                                            l_vec], v)
            return carry

        lax.fori_loop(0, BPW // 16, body, 0)

    def half(h, p, first=False, last=False):
        q = 1 - p
        if not last:
            wait_idx(q)
            issue_gathers(q)
        wait_gathers(p)
        if isinstance(h, int) and h + 2 <= HIST - 1:
            issue_idx(h + 2, p)
        elif not isinstance(h, int):
            issue_idx(h + 2, p)
        if not first:
            wait_out(p)
        transpose(p)
        issue_out(h, p)

    pltpu.sync_copy(idx_hbm.at[0, pl.ds(base, BPW)], idx_v.at[0])
    issue_gathers(0)
    issue_idx(1, 1)

    half(0, 0, first=True)
    half(1, 1, first=True)

    def body2(i, carry):
        h = 2 * i + 2
        half(h, 0)
        half(h + 1, 1)
        return carry

    lax.fori_loop(0, (HIST - 4) // 2, body2, 0)

    half(HIST - 2, 0)
    half(HIST - 1, 1, last=True)
    wait_out(0)
    wait_out(1)


def kernel(sets, E):
    out5 = _sc_gather(sets.T, E)
    return out5.transpose(2, 4, 0, 1, 3).reshape(BATCH, HIST, DIM)
